# Initial kernel scaffold; baseline (speedup 1.0000x reference)
import jax
import jax.numpy as jnp
from jax.experimental import pallas as pl


def kernel(x, W_enc, b_enc, W_base, alpha, b_dec):
    raise NotImplementedError("write your pallas kernel here")


# speedup vs baseline: 7.4084x; 7.4084x over previous
import functools

import jax
import jax.numpy as jnp
from jax.experimental import pallas as pl
from jax.experimental.pallas import tpu as pltpu

_INT_MIN = -2147483648
_INT_MAX = 2147483647


def _enc_kernel(x_ref, w_ref, b_ref, o_ref, *, nk):
    k = pl.program_id(1)

    @pl.when(k == 0)
    def _init():
        o_ref[...] = jnp.zeros_like(o_ref)

    o_ref[...] += jnp.dot(x_ref[...], w_ref[...],
                          preferred_element_type=jnp.float32)

    @pl.when(k == nk - 1)
    def _fin():
        o_ref[...] += b_ref[...]


def _encode(x2d, w2d, b2d, bn, bk):
    m, kdim = x2d.shape
    n = w2d.shape[1]
    nn, nk = n // bn, kdim // bk
    return pl.pallas_call(
        functools.partial(_enc_kernel, nk=nk),
        grid=(nn, nk),
        in_specs=[
            pl.BlockSpec((m, bk), lambda j, k: (0, k)),
            pl.BlockSpec((bk, bn), lambda j, k: (k, j)),
            pl.BlockSpec((1, bn), lambda j, k: (0, j)),
        ],
        out_specs=pl.BlockSpec((m, bn), lambda j, k: (0, j)),
        out_shape=jax.ShapeDtypeStruct((m, n), jnp.float32),
        compiler_params=pltpu.CompilerParams(
            dimension_semantics=("parallel", "arbitrary")),
    )(x2d, w2d, b2d)


def _sel_kernel(p_ref, z_ref, *, ktop):
    p = p_ref[...]
    bi = jax.lax.bitcast_convert_type(p, jnp.int32)
    key = jnp.where(bi >= 0, bi, jnp.int32(_INT_MIN) - bi)

    rows = p.shape[0]
    lo = jnp.full((rows, 1), _INT_MIN, jnp.int32)
    hi = jnp.full((rows, 1), _INT_MAX, jnp.int32)

    def body(_, lohi):
        lo, hi = lohi
        mid = (lo >> 1) + (hi >> 1) + (lo & hi & 1)
        cnt = jnp.sum((key >= mid).astype(jnp.int32), axis=1, keepdims=True)
        ge = cnt >= ktop
        return jnp.where(ge, mid, lo), jnp.where(ge, hi, mid)

    lo, hi = jax.lax.fori_loop(0, 32, body, (lo, hi))
    t = lo

    gt = key > t
    cnt_gt = jnp.sum(gt.astype(jnp.int32), axis=1, keepdims=True)
    eq = key == t
    need = ktop - cnt_gt
    idx = jax.lax.broadcasted_iota(jnp.int32, p.shape, 1)
    clo = jnp.zeros((rows, 1), jnp.int32)
    chi = jnp.full((rows, 1), p.shape[1], jnp.int32)

    def cbody(_, lohi):
        clo, chi = lohi
        mid = (clo + chi) // 2
        cnt = jnp.sum((eq & (idx < mid)).astype(jnp.int32), axis=1,
                      keepdims=True)
        ge = cnt >= need
        return jnp.where(ge, clo, mid + 1), jnp.where(ge, mid, chi)

    nbits = max(1, (p.shape[1]).bit_length())
    _, cut = jax.lax.fori_loop(0, nbits, cbody, (clo, chi))
    mask = gt | (eq & (idx < cut))
    z_ref[...] = jnp.where(mask, jnp.maximum(p, 0.0), 0.0)


def _select(pre, ktop, bm):
    m, n = pre.shape
    return pl.pallas_call(
        functools.partial(_sel_kernel, ktop=ktop),
        grid=(m // bm,),
        in_specs=[pl.BlockSpec((bm, n), lambda i: (i, 0))],
        out_specs=pl.BlockSpec((bm, n), lambda i: (i, 0)),
        out_shape=jax.ShapeDtypeStruct((m, n), jnp.float32),
        compiler_params=pltpu.CompilerParams(
            dimension_semantics=("parallel",)),
    )(pre)


def _dec_kernel(z_ref, w_ref, o_ref, *, ns, kb):
    s = pl.program_id(0)

    @pl.when(s == 0)
    def _init():
        o_ref[...] = jnp.zeros_like(o_ref)

    z = z_ref[...]
    w = w_ref[...]
    for k in range(kb):
        o_ref[:, k, :] += jnp.dot(z, w[k], preferred_element_type=jnp.float32)


def _decode(z16, w16, bs):
    m, n = z16.shape
    kb, _, d = w16.shape
    ns = n // bs
    return pl.pallas_call(
        functools.partial(_dec_kernel, ns=ns, kb=kb),
        grid=(ns,),
        in_specs=[
            pl.BlockSpec((m, bs), lambda s: (0, s)),
            pl.BlockSpec((kb, bs, d), lambda s: (0, s, 0)),
        ],
        out_specs=pl.BlockSpec((m, kb, d), lambda s: (0, 0, 0)),
        out_shape=jax.ShapeDtypeStruct((m, kb, d), jnp.float32),
        compiler_params=pltpu.CompilerParams(
            dimension_semantics=("arbitrary",)),
    )(z16, w16)


def _comb_kernel(zw_ref, a_ref, bd_ref, x_ref, xh_ref, lp_ref, *, kb):
    a = a_ref[...]
    zw = zw_ref[...]
    xh = jnp.zeros(xh_ref.shape, jnp.float32)
    for k in range(kb):
        xh += a[:, k][None, :, None] * zw[:, k, :][:, None, :]
    xh += bd_ref[...][None]
    xh_ref[...] = xh
    diff = xh - x_ref[...]
    lp_ref[...] = jnp.full(lp_ref.shape, jnp.sum(diff * diff))


def _combine(zw, alpha, b_dec, x, bm):
    m, kb, d = zw.shape
    t = alpha.shape[0]
    nm = m // bm
    return pl.pallas_call(
        functools.partial(_comb_kernel, kb=kb),
        grid=(nm,),
        in_specs=[
            pl.BlockSpec((bm, kb, d), lambda i: (i, 0, 0)),
            pl.BlockSpec((t, kb), lambda i: (0, 0)),
            pl.BlockSpec((t, d), lambda i: (0, 0)),
            pl.BlockSpec((bm, t, d), lambda i: (i, 0, 0)),
        ],
        out_specs=[
            pl.BlockSpec((bm, t, d), lambda i: (i, 0, 0)),
            pl.BlockSpec((1, 1, 128), lambda i: (i, 0, 0)),
        ],
        out_shape=[
            jax.ShapeDtypeStruct((m, t, d), jnp.float32),
            jax.ShapeDtypeStruct((nm, 1, 128), jnp.float32),
        ],
        compiler_params=pltpu.CompilerParams(
            dimension_semantics=("parallel",)),
    )(zw, alpha, b_dec, x)


def kernel(x, W_enc, b_enc, W_base, alpha, b_dec):
    b, t, din = x.shape
    dsae = W_enc.shape[-1]
    ktop = 64

    x2d = x.reshape(b, t * din)
    w2d = W_enc.reshape(t * din, dsae)

    pre = _encode(x2d, w2d, b_enc[None, :],
                  bn=min(1024, dsae), bk=min(1024, t * din))
    z = _select(pre, ktop, bm=min(256, b))

    z16 = z.astype(jnp.bfloat16)
    w16 = W_base.astype(jnp.bfloat16)
    zw = _decode(z16, w16, bs=min(1024, dsae))
    xh, lp = _combine(zw, alpha, b_dec, x, bm=min(256, b))

    loss = jnp.sum(lp[:, 0, 0]) / (b * t)
    return (loss, xh, z)


# speedup vs baseline: 8.4330x; 1.1383x over previous
import functools

import jax
import jax.numpy as jnp
from jax.experimental import pallas as pl
from jax.experimental.pallas import tpu as pltpu

_INT_MIN = -2147483648
_INT_MAX = 2147483647


def _enc_kernel(x_ref, w_ref, b_ref, o_ref, *, nk):
    k = pl.program_id(1)

    @pl.when(k == 0)
    def _init():
        o_ref[...] = jnp.zeros_like(o_ref)

    o_ref[...] += jnp.dot(x_ref[...], w_ref[...],
                          preferred_element_type=jnp.float32)

    @pl.when(k == nk - 1)
    def _fin():
        o_ref[...] += b_ref[...]


def _encode(x2d, w2d, b2d, bn, bk):
    m, kdim = x2d.shape
    n = w2d.shape[1]
    nn, nk = n // bn, kdim // bk
    return pl.pallas_call(
        functools.partial(_enc_kernel, nk=nk),
        grid=(nn, nk),
        in_specs=[
            pl.BlockSpec((m, bk), lambda j, k: (0, k)),
            pl.BlockSpec((bk, bn), lambda j, k: (k, j)),
            pl.BlockSpec((1, bn), lambda j, k: (0, j)),
        ],
        out_specs=pl.BlockSpec((m, bn), lambda j, k: (0, j)),
        out_shape=jax.ShapeDtypeStruct((m, n), jnp.float32),
        compiler_params=pltpu.CompilerParams(
            dimension_semantics=("parallel", "arbitrary")),
    )(x2d, w2d, b2d)


def _sel_kernel(p_ref, z_ref, z16_ref, *, ktop):
    p = p_ref[...]
    bi = jax.lax.bitcast_convert_type(p, jnp.int32)
    key = jnp.where(bi >= 0, bi, jnp.int32(_INT_MIN) - bi)

    rows = p.shape[0]
    lo = jnp.full((rows, 1), _INT_MIN, jnp.int32)
    hi = jnp.full((rows, 1), _INT_MAX, jnp.int32)

    def body(_, lohi):
        lo, hi = lohi
        mid = (lo >> 1) + (hi >> 1) + (lo & hi & 1)
        cnt = jnp.sum((key >= mid).astype(jnp.int32), axis=1, keepdims=True)
        ge = cnt >= ktop
        return jnp.where(ge, mid, lo), jnp.where(ge, hi, mid)

    lo, hi = jax.lax.fori_loop(0, 32, body, (lo, hi))
    t = lo

    gt = key > t
    cnt_gt = jnp.sum(gt.astype(jnp.int32), axis=1, keepdims=True)
    eq = key == t
    need = ktop - cnt_gt
    idx = jax.lax.broadcasted_iota(jnp.int32, p.shape, 1)
    clo = jnp.zeros((rows, 1), jnp.int32)
    chi = jnp.full((rows, 1), p.shape[1], jnp.int32)

    def cbody(_, lohi):
        clo, chi = lohi
        mid = (clo + chi) // 2
        cnt = jnp.sum((eq & (idx < mid)).astype(jnp.int32), axis=1,
                      keepdims=True)
        ge = cnt >= need
        return jnp.where(ge, clo, mid + 1), jnp.where(ge, mid, chi)

    nbits = max(1, (p.shape[1]).bit_length())
    _, cut = jax.lax.fori_loop(0, nbits, cbody, (clo, chi))
    mask = gt | (eq & (idx < cut))
    z = jnp.where(mask, jnp.maximum(p, 0.0), 0.0)
    z_ref[...] = z
    z16_ref[...] = z.astype(jnp.bfloat16)


def _select(pre, ktop, bm):
    m, n = pre.shape
    return pl.pallas_call(
        functools.partial(_sel_kernel, ktop=ktop),
        grid=(m // bm,),
        in_specs=[pl.BlockSpec((bm, n), lambda i: (i, 0))],
        out_specs=[pl.BlockSpec((bm, n), lambda i: (i, 0)),
                   pl.BlockSpec((bm, n), lambda i: (i, 0))],
        out_shape=[jax.ShapeDtypeStruct((m, n), jnp.float32),
                   jax.ShapeDtypeStruct((m, n), jnp.bfloat16)],
        compiler_params=pltpu.CompilerParams(
            dimension_semantics=("parallel",)),
    )(pre)


def _dec_kernel(z_ref, w_ref, o_ref, *, ns, kb):
    s = pl.program_id(0)

    @pl.when(s == 0)
    def _init():
        o_ref[...] = jnp.zeros_like(o_ref)

    z = z_ref[...]
    w = w_ref[...].astype(jnp.bfloat16)
    for k in range(kb):
        o_ref[:, k, :] += jnp.dot(z, w[k], preferred_element_type=jnp.float32)


def _decode(z16, w16, bs):
    m, n = z16.shape
    kb, _, d = w16.shape
    ns = n // bs
    return pl.pallas_call(
        functools.partial(_dec_kernel, ns=ns, kb=kb),
        grid=(ns,),
        in_specs=[
            pl.BlockSpec((m, bs), lambda s: (0, s)),
            pl.BlockSpec((kb, bs, d), lambda s: (0, s, 0)),
        ],
        out_specs=pl.BlockSpec((m, kb, d), lambda s: (0, 0, 0)),
        out_shape=jax.ShapeDtypeStruct((m, kb, d), jnp.float32),
        compiler_params=pltpu.CompilerParams(
            dimension_semantics=("arbitrary",)),
    )(z16, w16)


def _comb_kernel(zw_ref, a_ref, bd_ref, x_ref, xh_ref, lp_ref, *, kb):
    a = a_ref[...]
    zw = zw_ref[...]
    xh = jnp.zeros(xh_ref.shape, jnp.float32)
    for k in range(kb):
        xh += a[:, k][None, :, None] * zw[:, k, :][:, None, :]
    xh += bd_ref[...][None]
    xh_ref[...] = xh
    diff = xh - x_ref[...]
    lp_ref[...] = jnp.full(lp_ref.shape, jnp.sum(diff * diff))


def _combine(zw, alpha, b_dec, x, bm):
    m, kb, d = zw.shape
    t = alpha.shape[0]
    nm = m // bm
    return pl.pallas_call(
        functools.partial(_comb_kernel, kb=kb),
        grid=(nm,),
        in_specs=[
            pl.BlockSpec((bm, kb, d), lambda i: (i, 0, 0)),
            pl.BlockSpec((t, kb), lambda i: (0, 0)),
            pl.BlockSpec((t, d), lambda i: (0, 0)),
            pl.BlockSpec((bm, t, d), lambda i: (i, 0, 0)),
        ],
        out_specs=[
            pl.BlockSpec((bm, t, d), lambda i: (i, 0, 0)),
            pl.BlockSpec((1, 1, 128), lambda i: (i, 0, 0)),
        ],
        out_shape=[
            jax.ShapeDtypeStruct((m, t, d), jnp.float32),
            jax.ShapeDtypeStruct((nm, 1, 128), jnp.float32),
        ],
        compiler_params=pltpu.CompilerParams(
            dimension_semantics=("parallel",)),
    )(zw, alpha, b_dec, x)


def kernel(x, W_enc, b_enc, W_base, alpha, b_dec):
    b, t, din = x.shape
    dsae = W_enc.shape[-1]
    ktop = 64

    x2d = x.reshape(b, t * din)
    w2d = W_enc.reshape(t * din, dsae)

    pre = _encode(x2d, w2d, b_enc[None, :],
                  bn=min(2048, dsae), bk=min(1024, t * din))
    z, z16 = _select(pre, ktop, bm=min(256, b))

    zw = _decode(z16, W_base, bs=min(1024, dsae))
    xh, lp = _combine(zw, alpha, b_dec, x, bm=min(256, b))

    loss = jnp.sum(lp[:, 0, 0]) / (b * t)
    return (loss, xh, z)


# speedup vs baseline: 9.7143x; 1.1519x over previous
import functools

import jax
import jax.numpy as jnp
from jax.experimental import pallas as pl
from jax.experimental.pallas import tpu as pltpu

_INT_MIN = -2147483648
_INT_MAX = 2147483647


def _enc_kernel(x_ref, w_ref, b_ref, o_ref, *, nk):
    k = pl.program_id(1)

    @pl.when(k == 0)
    def _init():
        o_ref[...] = jnp.zeros_like(o_ref)

    o_ref[...] += jnp.dot(x_ref[...], w_ref[...],
                          preferred_element_type=jnp.float32)

    @pl.when(k == nk - 1)
    def _fin():
        o_ref[...] += b_ref[...]


def _encode(x2d, w2d, b2d, bn, bk):
    m, kdim = x2d.shape
    n = w2d.shape[1]
    nn, nk = n // bn, kdim // bk
    return pl.pallas_call(
        functools.partial(_enc_kernel, nk=nk),
        grid=(nn, nk),
        in_specs=[
            pl.BlockSpec((m, bk), lambda j, k: (0, k)),
            pl.BlockSpec((bk, bn), lambda j, k: (k, j)),
            pl.BlockSpec((1, bn), lambda j, k: (0, j)),
        ],
        out_specs=pl.BlockSpec((m, bn), lambda j, k: (0, j)),
        out_shape=jax.ShapeDtypeStruct((m, n), jnp.float32),
        compiler_params=pltpu.CompilerParams(
            dimension_semantics=("parallel", "arbitrary")),
    )(x2d, w2d, b2d)


def _avg(a, b):
    return (a >> 1) + (b >> 1) + (a & b & 1)


def _sel_kernel(p_ref, z_ref, z16_ref, *, ktop):
    p = p_ref[...]
    bi = jax.lax.bitcast_convert_type(p, jnp.int32)
    key = jnp.where(bi >= 0, bi, jnp.int32(_INT_MIN) - bi)

    rows, n = p.shape
    nlanes = 128
    lane_max = key[:, :nlanes]
    for c in range(1, n // nlanes):
        lane_max = jnp.maximum(lane_max, key[:, c * nlanes:(c + 1) * nlanes])
    if ktop <= nlanes:
        lo = jnp.min(lane_max, axis=1, keepdims=True)
    else:
        lo = jnp.full((rows, 1), _INT_MIN, jnp.int32)
    hi = jnp.max(lane_max, axis=1, keepdims=True) + 1

    def cond(lohi):
        lo, hi = lohi
        return jnp.any(lo + 1 < hi)

    def body(lohi):
        lo, hi = lohi
        m2 = _avg(lo, hi)
        m1 = _avg(lo, m2)
        m3 = _avg(m2, hi)
        c1 = jnp.sum((key >= m1).astype(jnp.int32), axis=1, keepdims=True)
        c2 = jnp.sum((key >= m2).astype(jnp.int32), axis=1, keepdims=True)
        c3 = jnp.sum((key >= m3).astype(jnp.int32), axis=1, keepdims=True)
        g1, g2, g3 = c1 >= ktop, c2 >= ktop, c3 >= ktop
        lo_n = jnp.where(g3, m3, jnp.where(g2, m2, jnp.where(g1, m1, lo)))
        hi_n = jnp.where(g3, hi, jnp.where(g2, m3, jnp.where(g1, m2, m1)))
        return lo_n, hi_n

    lo, hi = jax.lax.while_loop(cond, body, (lo, hi))
    t = lo

    gt = key > t
    cnt_gt = jnp.sum(gt.astype(jnp.int32), axis=1, keepdims=True)
    eq = key == t
    cnt_eq = jnp.sum(eq.astype(jnp.int32), axis=1, keepdims=True)
    need = ktop - cnt_gt
    idx = jax.lax.broadcasted_iota(jnp.int32, p.shape, 1)
    overflow = cnt_gt + cnt_eq > ktop
    clo = jnp.where(overflow, 0, n).astype(jnp.int32)
    chi = jnp.full((rows, 1), n, jnp.int32)

    def ccond(lohi):
        clo, chi = lohi
        return jnp.any(clo < chi)

    def cbody(lohi):
        clo, chi = lohi
        mid = (clo + chi) // 2
        cnt = jnp.sum((eq & (idx < mid)).astype(jnp.int32), axis=1,
                      keepdims=True)
        ge = cnt >= need
        return jnp.where(ge, clo, mid + 1), jnp.where(ge, mid, chi)

    _, cut = jax.lax.while_loop(ccond, cbody, (clo, chi))
    mask = gt | (eq & (idx < cut))
    z = jnp.where(mask, jnp.maximum(p, 0.0), 0.0)
    z_ref[...] = z
    z16_ref[...] = z.astype(jnp.bfloat16)


def _select(pre, ktop, bm):
    m, n = pre.shape
    return pl.pallas_call(
        functools.partial(_sel_kernel, ktop=ktop),
        grid=(m // bm,),
        in_specs=[pl.BlockSpec((bm, n), lambda i: (i, 0))],
        out_specs=[pl.BlockSpec((bm, n), lambda i: (i, 0)),
                   pl.BlockSpec((bm, n), lambda i: (i, 0))],
        out_shape=[jax.ShapeDtypeStruct((m, n), jnp.float32),
                   jax.ShapeDtypeStruct((m, n), jnp.bfloat16)],
        compiler_params=pltpu.CompilerParams(
            dimension_semantics=("parallel",)),
    )(pre)


def _dec_kernel(z_ref, w_ref, o_ref, *, ns, kb):
    s = pl.program_id(0)

    @pl.when(s == 0)
    def _init():
        o_ref[...] = jnp.zeros_like(o_ref)

    z = z_ref[...]
    w = w_ref[...].astype(jnp.bfloat16)
    for k in range(kb):
        o_ref[:, k, :] += jnp.dot(z, w[k], preferred_element_type=jnp.float32)


def _decode(z16, w16, bs):
    m, n = z16.shape
    kb, _, d = w16.shape
    ns = n // bs
    return pl.pallas_call(
        functools.partial(_dec_kernel, ns=ns, kb=kb),
        grid=(ns,),
        in_specs=[
            pl.BlockSpec((m, bs), lambda s: (0, s)),
            pl.BlockSpec((kb, bs, d), lambda s: (0, s, 0)),
        ],
        out_specs=pl.BlockSpec((m, kb, d), lambda s: (0, 0, 0)),
        out_shape=jax.ShapeDtypeStruct((m, kb, d), jnp.float32),
        compiler_params=pltpu.CompilerParams(
            dimension_semantics=("arbitrary",)),
    )(z16, w16)


def _comb_kernel(zw_ref, a_ref, bd_ref, x_ref, xh_ref, lp_ref, *, kb):
    a = a_ref[...]
    zw = zw_ref[...]
    xh = jnp.zeros(xh_ref.shape, jnp.float32)
    for k in range(kb):
        xh += a[:, k][None, :, None] * zw[:, k, :][:, None, :]
    xh += bd_ref[...][None]
    xh_ref[...] = xh
    diff = xh - x_ref[...]
    lp_ref[...] = jnp.full(lp_ref.shape, jnp.sum(diff * diff))


def _combine(zw, alpha, b_dec, x, bm):
    m, kb, d = zw.shape
    t = alpha.shape[0]
    nm = m // bm
    return pl.pallas_call(
        functools.partial(_comb_kernel, kb=kb),
        grid=(nm,),
        in_specs=[
            pl.BlockSpec((bm, kb, d), lambda i: (i, 0, 0)),
            pl.BlockSpec((t, kb), lambda i: (0, 0)),
            pl.BlockSpec((t, d), lambda i: (0, 0)),
            pl.BlockSpec((bm, t, d), lambda i: (i, 0, 0)),
        ],
        out_specs=[
            pl.BlockSpec((bm, t, d), lambda i: (i, 0, 0)),
            pl.BlockSpec((1, 1, 128), lambda i: (i, 0, 0)),
        ],
        out_shape=[
            jax.ShapeDtypeStruct((m, t, d), jnp.float32),
            jax.ShapeDtypeStruct((nm, 1, 128), jnp.float32),
        ],
        compiler_params=pltpu.CompilerParams(
            dimension_semantics=("parallel",)),
    )(zw, alpha, b_dec, x)


def kernel(x, W_enc, b_enc, W_base, alpha, b_dec):
    b, t, din = x.shape
    dsae = W_enc.shape[-1]
    ktop = 64

    x2d = x.reshape(b, t * din)
    w2d = W_enc.reshape(t * din, dsae)

    pre = _encode(x2d, w2d, b_enc[None, :],
                  bn=min(2048, dsae), bk=min(1024, t * din))
    z, z16 = _select(pre, ktop, bm=min(256, b))

    zw = _decode(z16, W_base, bs=min(1024, dsae))
    xh, lp = _combine(zw, alpha, b_dec, x, bm=min(256, b))

    loss = jnp.sum(lp[:, 0, 0]) / (b * t)
    return (loss, xh, z)


# speedup vs baseline: 9.8644x; 1.0155x over previous
import functools

import jax
import jax.numpy as jnp
from jax.experimental import pallas as pl
from jax.experimental.pallas import tpu as pltpu

_INT_MIN = -2147483648
_INT_MAX = 2147483647


def _enc_kernel(x_ref, w_ref, b_ref, o_ref, *, nk):
    k = pl.program_id(1)

    @pl.when(k == 0)
    def _init():
        o_ref[...] = jnp.zeros_like(o_ref)

    o_ref[...] += jnp.dot(x_ref[...], w_ref[...],
                          preferred_element_type=jnp.float32)

    @pl.when(k == nk - 1)
    def _fin():
        o_ref[...] += b_ref[...]


def _encode(x2d, w2d, b2d, bn, bk):
    m, kdim = x2d.shape
    n = w2d.shape[1]
    nn, nk = n // bn, kdim // bk
    return pl.pallas_call(
        functools.partial(_enc_kernel, nk=nk),
        grid=(nn, nk),
        in_specs=[
            pl.BlockSpec((m, bk), lambda j, k: (0, k)),
            pl.BlockSpec((bk, bn), lambda j, k: (k, j)),
            pl.BlockSpec((1, bn), lambda j, k: (0, j)),
        ],
        out_specs=pl.BlockSpec((m, bn), lambda j, k: (0, j)),
        out_shape=jax.ShapeDtypeStruct((m, n), jnp.float32),
        compiler_params=pltpu.CompilerParams(
            dimension_semantics=("parallel", "arbitrary")),
    )(x2d, w2d, b2d)


def _avg(a, b):
    return (a >> 1) + (b >> 1) + (a & b & 1)


def _sel_kernel(p_ref, z_ref, z16_ref, *, ktop):
    p = p_ref[...]
    bi = jax.lax.bitcast_convert_type(p, jnp.int32)
    key = jnp.where(bi >= 0, bi, jnp.int32(_INT_MIN) - bi)

    rows, n = p.shape
    nlanes = 128
    lane_max = key[:, :nlanes]
    for c in range(1, n // nlanes):
        lane_max = jnp.maximum(lane_max, key[:, c * nlanes:(c + 1) * nlanes])
    if ktop <= nlanes:
        lo = jnp.min(lane_max, axis=1, keepdims=True)
    else:
        lo = jnp.full((rows, 1), _INT_MIN, jnp.int32)
    hi = jnp.max(lane_max, axis=1, keepdims=True) + 1

    def cond(lohi):
        lo, hi = lohi
        return jnp.any(lo + 1 < hi)

    def body(lohi):
        lo, hi = lohi
        m2 = _avg(lo, hi)
        m1 = _avg(lo, m2)
        m3 = _avg(m2, hi)
        c1 = jnp.sum((key >= m1).astype(jnp.int32), axis=1, keepdims=True)
        c2 = jnp.sum((key >= m2).astype(jnp.int32), axis=1, keepdims=True)
        c3 = jnp.sum((key >= m3).astype(jnp.int32), axis=1, keepdims=True)
        g1, g2, g3 = c1 >= ktop, c2 >= ktop, c3 >= ktop
        lo_n = jnp.where(g3, m3, jnp.where(g2, m2, jnp.where(g1, m1, lo)))
        hi_n = jnp.where(g3, hi, jnp.where(g2, m3, jnp.where(g1, m2, m1)))
        return lo_n, hi_n

    lo, hi = jax.lax.while_loop(cond, body, (lo, hi))
    t = lo

    gt = key > t
    cnt_gt = jnp.sum(gt.astype(jnp.int32), axis=1, keepdims=True)
    eq = key == t
    cnt_eq = jnp.sum(eq.astype(jnp.int32), axis=1, keepdims=True)
    need = ktop - cnt_gt
    idx = jax.lax.broadcasted_iota(jnp.int32, p.shape, 1)
    overflow = cnt_gt + cnt_eq > ktop
    clo = jnp.where(overflow, 0, n).astype(jnp.int32)
    chi = jnp.full((rows, 1), n, jnp.int32)

    def ccond(lohi):
        clo, chi = lohi
        return jnp.any(clo < chi)

    def cbody(lohi):
        clo, chi = lohi
        mid = (clo + chi) // 2
        cnt = jnp.sum((eq & (idx < mid)).astype(jnp.int32), axis=1,
                      keepdims=True)
        ge = cnt >= need
        return jnp.where(ge, clo, mid + 1), jnp.where(ge, mid, chi)

    _, cut = jax.lax.while_loop(ccond, cbody, (clo, chi))
    mask = gt | (eq & (idx < cut))
    z = jnp.where(mask, jnp.maximum(p, 0.0), 0.0)
    z_ref[...] = z
    z16_ref[...] = z.astype(jnp.bfloat16)


def _select(pre, ktop, bm):
    m, n = pre.shape
    return pl.pallas_call(
        functools.partial(_sel_kernel, ktop=ktop),
        grid=(m // bm,),
        in_specs=[pl.BlockSpec((bm, n), lambda i: (i, 0))],
        out_specs=[pl.BlockSpec((bm, n), lambda i: (i, 0)),
                   pl.BlockSpec((bm, n), lambda i: (i, 0))],
        out_shape=[jax.ShapeDtypeStruct((m, n), jnp.float32),
                   jax.ShapeDtypeStruct((m, n), jnp.bfloat16)],
        compiler_params=pltpu.CompilerParams(
            dimension_semantics=("parallel",)),
    )(pre)


def _dec_kernel(z_ref, w_ref, o_ref, *, ns, kb):
    s = pl.program_id(0)

    @pl.when(s == 0)
    def _init():
        o_ref[...] = jnp.zeros_like(o_ref)

    z = z_ref[...]
    w = w_ref[...].astype(jnp.bfloat16)
    for k in range(kb):
        o_ref[:, k, :] += jnp.dot(z, w[k], preferred_element_type=jnp.float32)


def _decode(z16, w16, bs):
    m, n = z16.shape
    kb, _, d = w16.shape
    ns = n // bs
    return pl.pallas_call(
        functools.partial(_dec_kernel, ns=ns, kb=kb),
        grid=(ns,),
        in_specs=[
            pl.BlockSpec((m, bs), lambda s: (0, s)),
            pl.BlockSpec((kb, bs, d), lambda s: (0, s, 0)),
        ],
        out_specs=pl.BlockSpec((m, kb, d), lambda s: (0, 0, 0)),
        out_shape=jax.ShapeDtypeStruct((m, kb, d), jnp.float32),
        compiler_params=pltpu.CompilerParams(
            dimension_semantics=("arbitrary",)),
    )(z16, w16)


def _comb_kernel(zw_ref, a_ref, bd_ref, x_ref, xh_ref, lp_ref, *, kb):
    a = a_ref[...]
    zw = zw_ref[...]
    xh = jnp.zeros(xh_ref.shape, jnp.float32)
    for k in range(kb):
        xh += a[:, k][None, :, None] * zw[:, k, :][:, None, :]
    xh += bd_ref[...][None]
    xh_ref[...] = xh
    diff = xh - x_ref[...]
    lp_ref[...] = jnp.full(lp_ref.shape, jnp.sum(diff * diff))


def _combine(zw, alpha, b_dec, x, bm):
    m, kb, d = zw.shape
    t = alpha.shape[0]
    nm = m // bm
    return pl.pallas_call(
        functools.partial(_comb_kernel, kb=kb),
        grid=(nm,),
        in_specs=[
            pl.BlockSpec((bm, kb, d), lambda i: (i, 0, 0)),
            pl.BlockSpec((t, kb), lambda i: (0, 0)),
            pl.BlockSpec((t, d), lambda i: (0, 0)),
            pl.BlockSpec((bm, t, d), lambda i: (i, 0, 0)),
        ],
        out_specs=[
            pl.BlockSpec((bm, t, d), lambda i: (i, 0, 0)),
            pl.BlockSpec((1, 1, 128), lambda i: (i, 0, 0)),
        ],
        out_shape=[
            jax.ShapeDtypeStruct((m, t, d), jnp.float32),
            jax.ShapeDtypeStruct((nm, 1, 128), jnp.float32),
        ],
        compiler_params=pltpu.CompilerParams(
            dimension_semantics=("parallel",)),
    )(zw, alpha, b_dec, x)


def kernel(x, W_enc, b_enc, W_base, alpha, b_dec):
    b, t, din = x.shape
    dsae = W_enc.shape[-1]
    ktop = 64

    x2d = x.reshape(b, t * din)
    w2d = W_enc.reshape(t * din, dsae)

    pre = _encode(x2d, w2d, b_enc[None, :],
                  bn=min(512, dsae), bk=t * din)
    z, z16 = _select(pre, ktop, bm=min(256, b))

    zw = _decode(z16, W_base, bs=min(1024, dsae))
    xh, lp = _combine(zw, alpha, b_dec, x, bm=min(256, b))

    loss = jnp.sum(lp[:, 0, 0]) / (b * t)
    return (loss, xh, z)
